# Initial kernel scaffold; baseline (speedup 1.0000x reference)
import jax
import jax.numpy as jnp
from jax.experimental import pallas as pl


def kernel(input_pcd, query_points, params):
    raise NotImplementedError("write your pallas kernel here")


# speedup vs baseline: 13.3768x; 13.3768x over previous
import functools

import jax
import jax.numpy as jnp
from jax import lax
from jax.experimental import pallas as pl
from jax.experimental.pallas import tpu as pltpu
from jax.experimental.pallas import tpu_sc as plsc

KNN = 10
KP = 16
BN_EPS = 1e-5
def _dot(x, w):
  return jax.lax.dot_general(
      x, w, (((1,), (0,)), ((), ())),
      preferred_element_type=jnp.float32)


def _knn_body(q_ref, pt_ref, idx_ref):
  q = q_ref[0]
  pt = pt_ref[0]
  b = pl.program_id(0)
  n = pt.shape[1]
  q2 = jnp.sum(q * q, axis=1, keepdims=True)
  p2 = jnp.sum(pt * pt, axis=0, keepdims=True)
  d2 = q2 + p2 - 2.0 * _dot(q, pt)
  lane = jax.lax.broadcasted_iota(jnp.int32, d2.shape, 1)
  cols = []
  for _ in range(KNN):
    m = jnp.min(d2, axis=1, keepdims=True)
    cand = jnp.where(d2 == m, lane, n)
    idx = jnp.min(cand, axis=1, keepdims=True)
    d2 = jnp.where(cand == idx, jnp.float32(jnp.inf), d2)
    cols.append(idx)
  cols += [jnp.zeros_like(cols[0])] * (KP - KNN)
  idx_ref[0] = jnp.concatenate(cols, axis=1) + b * n


def _knn_topk(q_rows, p_t):
  B, M, _ = q_rows.shape
  N = p_t.shape[2]
  TM = 128
  return pl.pallas_call(
      _knn_body,
      grid=(B, M // TM),
      in_specs=[
          pl.BlockSpec((1, TM, 3), lambda b, i: (b, i, 0)),
          pl.BlockSpec((1, 3, N), lambda b, i: (b, 0, 0)),
      ],
      out_specs=pl.BlockSpec((1, TM, KP), lambda b, i: (b, i, 0)),
      out_shape=jax.ShapeDtypeStruct((B, M, KP), jnp.int32),
  )(q_rows, p_t)


def _gather_sc(px, py, pz, idxg):
  BN = px.shape[0]
  R = idxg.shape[0]
  info = plsc.get_sparse_core_info()
  nw = info.num_cores * info.num_subcores
  per_w = R // nw
  mesh = plsc.VectorSubcoreMesh(core_axis_name="c", subcore_axis_name="s")
  fvec = jax.ShapeDtypeStruct((R,), jnp.float32)

  @functools.partial(
      pl.kernel, mesh=mesh,
      out_type=[fvec, fvec, fvec],
      compiler_params=pltpu.CompilerParams(needs_layout_passes=False),
      scratch_types=[
          pltpu.VMEM((BN,), jnp.float32),
          pltpu.VMEM((BN,), jnp.float32),
          pltpu.VMEM((BN,), jnp.float32),
          pltpu.VMEM((per_w,), jnp.int32),
          pltpu.VMEM((per_w,), jnp.float32),
          pltpu.VMEM((per_w,), jnp.float32),
          pltpu.VMEM((per_w,), jnp.float32),
      ])
  def gk(px_hbm, py_hbm, pz_hbm, idx_hbm, outx, outy, outz,
         tabx, taby, tabz, idx_v, ox, oy, oz):
    wid = lax.axis_index("s") * info.num_cores + lax.axis_index("c")
    base = wid * per_w
    pltpu.sync_copy(px_hbm, tabx)
    pltpu.sync_copy(py_hbm, taby)
    pltpu.sync_copy(pz_hbm, tabz)
    pltpu.sync_copy(idx_hbm.at[pl.ds(base, per_w)], idx_v)

    def body(i, carry):
      sl = pl.ds(i * 16, 16)
      rid = idx_v[sl]
      ox[sl] = plsc.load_gather(tabx, [rid])
      oy[sl] = plsc.load_gather(taby, [rid])
      oz[sl] = plsc.load_gather(tabz, [rid])
      return carry

    lax.fori_loop(0, per_w // 16, body, 0)
    pltpu.sync_copy(ox, outx.at[pl.ds(base, per_w)])
    pltpu.sync_copy(oy, outy.at[pl.ds(base, per_w)])
    pltpu.sync_copy(oz, outz.at[pl.ds(base, per_w)])

  return gk(px, py, pz, idxg)


def _kmask(tr):
  row = jax.lax.broadcasted_iota(jnp.int32, (tr, 1), 0)
  return (row & (KP - 1)) < KNN


def _acc_stats(st_ref, y, mask, step):
  ym = jnp.where(mask, y, 0.0)
  s = jnp.sum(ym, axis=0, keepdims=True)
  sq = jnp.sum(ym * ym, axis=0, keepdims=True)
  st = jnp.concatenate([s, sq], axis=0)

  @pl.when(step == 0)
  def _():
    st_ref[...] = jnp.zeros_like(st_ref)

  st_ref[...] += st


def _stageA_body(knn_ref, qb_ref, wl_ref, wq_ref, b0_ref,
                 loc_ref, z_ref, st_ref):
  tr = knn_ref.shape[0]
  local = qb_ref[...] - knn_ref[...]
  loc_ref[...] = local
  y = _dot(local, wl_ref[...]) + _dot(qb_ref[...], wq_ref[...]) + b0_ref[...]
  z_ref[...] = y
  _acc_stats(st_ref, y, _kmask(tr), pl.program_id(0))


def _stageA(knn_rows, q_b, w0l, w0q, b0):
  R = knn_rows.shape[0]
  TR = 2048
  co = w0l.shape[1]
  return pl.pallas_call(
      _stageA_body,
      grid=(R // TR,),
      in_specs=[
          pl.BlockSpec((TR, 3), lambda i: (i, 0)),
          pl.BlockSpec((TR, 3), lambda i: (i, 0)),
          pl.BlockSpec(w0l.shape, lambda i: (0, 0)),
          pl.BlockSpec(w0q.shape, lambda i: (0, 0)),
          pl.BlockSpec((1, co), lambda i: (0, 0)),
      ],
      out_specs=[
          pl.BlockSpec((TR, 3), lambda i: (i, 0)),
          pl.BlockSpec((TR, co), lambda i: (i, 0)),
          pl.BlockSpec((2, co), lambda i: (0, 0)),
      ],
      out_shape=[
          jax.ShapeDtypeStruct((R, 3), jnp.float32),
          jax.ShapeDtypeStruct((R, co), jnp.float32),
          jax.ShapeDtypeStruct((2, co), jnp.float32),
      ],
  )(knn_rows, q_b, w0l, w0q, b0)


def _bn_mm_body(z_ref, a_ref, c_ref, w_ref, b_ref, out_ref, st_ref, *,
                with_stats):
  tr = z_ref.shape[0]
  x = z_ref[...] * a_ref[...] + c_ref[...]
  x = jnp.where(x >= 0, x, 0.2 * x)
  y = _dot(x, w_ref[...]) + b_ref[...]
  out_ref[...] = y
  if with_stats:
    _acc_stats(st_ref, y, _kmask(tr), pl.program_id(0))


def _bn_mm(z, aff, w, b, with_stats=True):
  R, ci = z.shape
  co = w.shape[1]
  TR = 2048
  a, c = aff
  body = functools.partial(_bn_mm_body, with_stats=with_stats)
  out_specs = [pl.BlockSpec((TR, co), lambda i: (i, 0)),
               pl.BlockSpec((2, co), lambda i: (0, 0))]
  out_shape = [jax.ShapeDtypeStruct((R, co), jnp.float32),
               jax.ShapeDtypeStruct((2, co), jnp.float32)]
  return pl.pallas_call(
      body,
      grid=(R // TR,),
      in_specs=[
          pl.BlockSpec((TR, ci), lambda i: (i, 0)),
          pl.BlockSpec((1, ci), lambda i: (0, 0)),
          pl.BlockSpec((1, ci), lambda i: (0, 0)),
          pl.BlockSpec((ci, co), lambda i: (0, 0)),
          pl.BlockSpec((1, co), lambda i: (0, 0)),
      ],
      out_specs=out_specs,
      out_shape=out_shape,
  )(z, a, c, w, b)


def _stageD_body(z3_ref, a_ref, c_ref, w3_ref, b3_ref, loc_ref, qb_ref,
                 wl_ref, wq_ref, wd_ref, wf_ref, wp_ref, b0_ref,
                 out_ref, st_ref):
  tr = z3_ref.shape[0]
  x3 = z3_ref[...] * a_ref[...] + c_ref[...]
  x3 = jnp.where(x3 >= 0, x3, 0.2 * x3)
  feat = _dot(x3, w3_ref[...]) + b3_ref[...]
  mask = _kmask(tr)
  fm = jnp.where(mask, feat, -jnp.inf)
  g = tr // KP
  fg = jnp.max(fm.reshape(g, KP, feat.shape[1]), axis=1)
  pf = jnp.broadcast_to(fg[:, None, :], (g, KP, feat.shape[1]))
  pf = pf.reshape(tr, feat.shape[1])
  local = loc_ref[...]
  kd = jnp.sqrt(jnp.sum(local * local, axis=1, keepdims=True))
  y = (_dot(local, wl_ref[...]) + _dot(qb_ref[...], wq_ref[...])
       + kd * wd_ref[...] + _dot(feat, wf_ref[...]) + _dot(pf, wp_ref[...])
       + b0_ref[...])
  out_ref[...] = y
  _acc_stats(st_ref, y, mask, pl.program_id(0))


def _stageD(z3, aff, w3, b3, local, q_b, wl, wq, wd, wf, wp, b0):
  R, ci = z3.shape
  co = wl.shape[1]
  TR = 2048
  a, c = aff
  return pl.pallas_call(
      _stageD_body,
      grid=(R // TR,),
      in_specs=[
          pl.BlockSpec((TR, ci), lambda i: (i, 0)),
          pl.BlockSpec((1, ci), lambda i: (0, 0)),
          pl.BlockSpec((1, ci), lambda i: (0, 0)),
          pl.BlockSpec(w3.shape, lambda i: (0, 0)),
          pl.BlockSpec((1, w3.shape[1]), lambda i: (0, 0)),
          pl.BlockSpec((TR, 3), lambda i: (i, 0)),
          pl.BlockSpec((TR, 3), lambda i: (i, 0)),
          pl.BlockSpec(wl.shape, lambda i: (0, 0)),
          pl.BlockSpec(wq.shape, lambda i: (0, 0)),
          pl.BlockSpec(wd.shape, lambda i: (0, 0)),
          pl.BlockSpec(wf.shape, lambda i: (0, 0)),
          pl.BlockSpec(wp.shape, lambda i: (0, 0)),
          pl.BlockSpec((1, co), lambda i: (0, 0)),
      ],
      out_specs=[
          pl.BlockSpec((TR, co), lambda i: (i, 0)),
          pl.BlockSpec((2, co), lambda i: (0, 0)),
      ],
      out_shape=[
          jax.ShapeDtypeStruct((R, co), jnp.float32),
          jax.ShapeDtypeStruct((2, co), jnp.float32),
      ],
  )(z3, a, c, w3, b3, local, q_b, wl, wq, wd, wf, wp, b0)


def _stageG_body(z_ref, a_ref, c_ref, w_ref, b_ref, loc_ref, out_ref):
  tr = z_ref.shape[0]
  x = z_ref[...] * a_ref[...] + c_ref[...]
  x = jnp.where(x >= 0, x, 0.2 * x)
  wlog = _dot(x, w_ref[...]) + b_ref[...]
  g = tr // KP
  w3 = wlog.reshape(g, KP, 1)
  mask = (jax.lax.broadcasted_iota(jnp.int32, (1, KP, 1), 1) < KNN)
  mx = jnp.max(jnp.where(mask, w3, -jnp.inf), axis=1, keepdims=True)
  e = jnp.where(mask, jnp.exp(w3 - mx), 0.0)
  s = jnp.sum(e, axis=1, keepdims=True)
  w = e / s
  loc3 = loc_ref[...].reshape(g, KP, 3)
  vec = jnp.sum(w * loc3, axis=1)
  out_ref[...] = jnp.sqrt(jnp.sum(vec * vec, axis=1, keepdims=True))


def _stageG(z, aff, w, b, local):
  R, ci = z.shape
  TR = 2048
  a, c = aff
  G = TR // KP
  return pl.pallas_call(
      _stageG_body,
      grid=(R // TR,),
      in_specs=[
          pl.BlockSpec((TR, ci), lambda i: (i, 0)),
          pl.BlockSpec((1, ci), lambda i: (0, 0)),
          pl.BlockSpec((1, ci), lambda i: (0, 0)),
          pl.BlockSpec((ci, 1), lambda i: (0, 0)),
          pl.BlockSpec((1, 1), lambda i: (0, 0)),
          pl.BlockSpec((TR, 3), lambda i: (i, 0)),
      ],
      out_specs=pl.BlockSpec((G, 1), lambda i: (i, 0)),
      out_shape=jax.ShapeDtypeStruct((R // KP, 1), jnp.float32),
  )(z, a, c, w, b, local)


_R_REAL = None


def _aff_from_stats(st, g, beta, n_real):
  mean = st[0] / n_real
  var = st[1] / n_real - mean * mean
  a = g / jnp.sqrt(var + BN_EPS)
  c = beta - mean * a
  return a[None, :], c[None, :]


def kernel(input_pcd, query_points, params):
  B, N, _ = input_pcd.shape
  M = query_points.shape[2]
  R = B * M * KP
  n_real = jnp.float32(B * M * KNN)

  q_rows = jnp.transpose(query_points, (0, 2, 1))
  p_t = jnp.transpose(input_pcd, (0, 2, 1))

  idx = _knn_topk(q_rows, p_t)
  idxg = idx.reshape(R)

  pf = input_pcd.reshape(B * N, 3)
  gx, gy, gz = _gather_sc(pf[:, 0], pf[:, 1], pf[:, 2], idxg)
  knn_rows = jnp.stack([gx, gy, gz], axis=1)

  q_b = jnp.broadcast_to(q_rows[:, :, None, :], (B, M, KP, 3)).reshape(R, 3)

  tw = lambda name: jnp.transpose(params[name])
  row = lambda name: params[name][None, :]

  w0t = tw('patch_W0')
  local, z1, st1 = _stageA(knn_rows, q_b, w0t[0:3], w0t[3:6], row('patch_b0'))

  aff1 = _aff_from_stats(st1, params['patch_g0'], params['patch_beta0'], n_real)
  z2, st2 = _bn_mm(z1, aff1, tw('patch_W1'), row('patch_b1'))
  aff2 = _aff_from_stats(st2, params['patch_g1'], params['patch_beta1'], n_real)
  z3, st3 = _bn_mm(z2, aff2, tw('patch_W2'), row('patch_b2'))
  aff3 = _aff_from_stats(st3, params['patch_g2'], params['patch_beta2'], n_real)

  a0t = tw('attn_W0')
  z1a, st1a = _stageD(
      z3, aff3, tw('patch_W3'), row('patch_b3'), local, q_b,
      a0t[0:3], a0t[3:6], a0t[6:7], a0t[7:135], a0t[135:263], row('attn_b0'))

  aff1a = _aff_from_stats(st1a, params['attn_g0'], params['attn_beta0'], n_real)
  z2a, st2a = _bn_mm(z1a, aff1a, tw('attn_W1'), row('attn_b1'))
  aff2a = _aff_from_stats(st2a, params['attn_g1'], params['attn_beta1'], n_real)
  z3a, st3a = _bn_mm(z2a, aff2a, tw('attn_W2'), row('attn_b2'))
  aff3a = _aff_from_stats(st3a, params['attn_g2'], params['attn_beta2'], n_real)

  udf = _stageG(z3a, aff3a, tw('attn_W3'), row('attn_b3'), local)
  return udf.reshape(B, M)
